# baseline (speedup 1.0000x reference)
import functools

import jax
import jax.numpy as jnp
from jax import lax
from jax.experimental import pallas as pl
from jax.experimental.pallas import tpu as pltpu
from jax.experimental.pallas import tpu_sc as plsc

N = 10000
E = 320000
D = 128
G = 64
NC = 2
NS = 16
NW = NC * NS

CH = 128
NCH = 80
NPAIR = NCH // 2
EPW = NCH * CH
EPAD = NW * EPW
NP = 10112

RPT = 624
RTAIL = N - NS * RPT
RTOFF = NS * RPT

BLK = 2000
NB = N // BLK

_mesh = plsc.VectorSubcoreMesh(core_axis_name="c", subcore_axis_name="s")


def _wait_rows(sem, g_hbm, rows):
    pltpu.make_async_copy(g_hbm.at[pl.ds(0, CH)], rows, sem).wait()


@functools.partial(
    pl.kernel,
    out_type=jax.ShapeDtypeStruct((NC, N, D), jnp.float32),
    mesh=_mesh,
    scratch_types=[
        pltpu.VMEM((NCH, CH), jnp.int32),
        pltpu.VMEM((CH, D), jnp.float32),
        pltpu.VMEM_SHARED((NP, D), jnp.float32),
        pltpu.SemaphoreType.DMA,
        pltpu.SemaphoreType.DMA,
    ],
)
def _deg_pass(dst2_hbm, zD_hbm, ones_hbm, out_hbm,
              dstb_v, ones_v, acc_sh, sem_a, sem_b):
    c = lax.axis_index("c")
    s = lax.axis_index("s")
    w = c * NS + s
    rbase = s * RPT
    pltpu.sync_copy(zD_hbm.at[pl.ds(rbase, RPT)], acc_sh.at[pl.ds(rbase, RPT)])

    @pl.when(s == NS - 1)
    def _():
        pltpu.sync_copy(zD_hbm.at[pl.ds(RTOFF, RTAIL)],
                        acc_sh.at[pl.ds(RTOFF, RTAIL)])

    pltpu.sync_copy(ones_hbm, ones_v)
    pltpu.sync_copy(dst2_hbm.at[pl.ds(w * NCH, NCH)], dstb_v)
    plsc.subcore_barrier()

    def body(j, carry):
        @pl.when(j > 0)
        def _():
            _wait_rows(sem_a, zD_hbm, ones_v)
        pltpu.async_copy(ones_v, acc_sh.at[dstb_v.at[2 * j]], sem_a, add=True)

        @pl.when(j > 0)
        def _():
            _wait_rows(sem_b, zD_hbm, ones_v)
        pltpu.async_copy(ones_v, acc_sh.at[dstb_v.at[2 * j + 1]], sem_b,
                         add=True)
        return carry

    lax.fori_loop(0, NPAIR, body, 0)
    _wait_rows(sem_a, zD_hbm, ones_v)
    _wait_rows(sem_b, zD_hbm, ones_v)
    plsc.subcore_barrier()
    pltpu.sync_copy(acc_sh.at[pl.ds(rbase, RPT)], out_hbm.at[c, pl.ds(rbase, RPT)])

    @pl.when(s == NS - 1)
    def _():
        pltpu.sync_copy(acc_sh.at[pl.ds(RTOFF, RTAIL)],
                        out_hbm.at[c, pl.ds(RTOFF, RTAIL)])


@functools.partial(
    pl.kernel,
    out_type=jax.ShapeDtypeStruct((NC, N, D), jnp.float32),
    mesh=_mesh,
    scratch_types=[
        pltpu.VMEM((NCH, CH), jnp.int32),
        pltpu.VMEM((CH,), jnp.int32),
        pltpu.VMEM((CH,), jnp.int32),
        pltpu.VMEM((CH, D), jnp.float32),
        pltpu.VMEM((CH, D), jnp.float32),
        pltpu.VMEM_SHARED((NP, D), jnp.float32),
        pltpu.SemaphoreType.DMA,
        pltpu.SemaphoreType.DMA,
        pltpu.SemaphoreType.DMA,
        pltpu.SemaphoreType.DMA,
        pltpu.SemaphoreType.DMA,
        pltpu.SemaphoreType.DMA,
    ],
)
def _edge_pass(g_hbm, src_hbm, dst2_hbm, zD_hbm, out_hbm,
               dstb_v, srcv_a, srcv_b, rows_a, rows_b, acc_sh,
               isem_a, isem_b, gsem_a, gsem_b, ssem_a, ssem_b):
    c = lax.axis_index("c")
    s = lax.axis_index("s")
    w = c * NS + s
    rbase = s * RPT
    ebase = w * EPW
    pltpu.sync_copy(zD_hbm.at[pl.ds(rbase, RPT)], acc_sh.at[pl.ds(rbase, RPT)])

    @pl.when(s == NS - 1)
    def _():
        pltpu.sync_copy(zD_hbm.at[pl.ds(RTOFF, RTAIL)],
                        acc_sh.at[pl.ds(RTOFF, RTAIL)])

    pltpu.sync_copy(dst2_hbm.at[pl.ds(w * NCH, NCH)], dstb_v)
    plsc.subcore_barrier()

    def idx(chunk, srcv, sem):
        pltpu.async_copy(src_hbm.at[pl.ds(ebase + chunk * CH, CH)], srcv, sem)

    def wait_idx(sem, srcv):
        pltpu.make_async_copy(src_hbm.at[pl.ds(0, CH)], srcv, sem).wait()

    idx(0, srcv_a, isem_a)
    idx(1, srcv_b, isem_b)

    def body(j, carry):
        @pl.when(j > 0)
        def _():
            _wait_rows(ssem_a, g_hbm, rows_a)
        wait_idx(isem_a, srcv_a)
        pltpu.async_copy(g_hbm.at[srcv_a], rows_a, gsem_a)

        @pl.when(j > 0)
        def _():
            _wait_rows(gsem_b, g_hbm, rows_b)
            pltpu.async_copy(rows_b, acc_sh.at[dstb_v.at[2 * j - 1]], ssem_b,
                             add=True)
            idx(2 * j + 1, srcv_b, isem_b)
            _wait_rows(ssem_b, g_hbm, rows_b)

        wait_idx(isem_b, srcv_b)
        pltpu.async_copy(g_hbm.at[srcv_b], rows_b, gsem_b)
        _wait_rows(gsem_a, g_hbm, rows_a)
        pltpu.async_copy(rows_a, acc_sh.at[dstb_v.at[2 * j]], ssem_a, add=True)

        @pl.when(j < NPAIR - 1)
        def _():
            idx(2 * j + 2, srcv_a, isem_a)
        return carry

    lax.fori_loop(0, NPAIR, body, 0)
    _wait_rows(gsem_b, g_hbm, rows_b)
    pltpu.async_copy(rows_b, acc_sh.at[dstb_v.at[NCH - 1]], ssem_b, add=True)
    _wait_rows(ssem_a, g_hbm, rows_a)
    _wait_rows(ssem_b, g_hbm, rows_b)
    plsc.subcore_barrier()
    pltpu.sync_copy(acc_sh.at[pl.ds(rbase, RPT)], out_hbm.at[c, pl.ds(rbase, RPT)])

    @pl.when(s == NS - 1)
    def _():
        pltpu.sync_copy(acc_sh.at[pl.ds(RTOFF, RTAIL)],
                        out_hbm.at[c, pl.ds(RTOFF, RTAIL)])


def _dis_from(degp_ref):
    deg = degp_ref[0, :, 0:1] + degp_ref[1, :, 0:1] + 1.0
    return lax.rsqrt(deg)


def _g1_body(x_ref, w_ref, degp_ref, o_ref):
    dis = _dis_from(degp_ref)
    h = lax.dot_general(x_ref[...], w_ref[...], (((1,), (1,)), ((), ())),
                        preferred_element_type=jnp.float32,
                        precision=lax.Precision.HIGHEST)
    o_ref[...] = h * dis


def _g2_body(p_ref, g1_ref, degp_ref, w_ref, o_ref):
    dis = _dis_from(degp_ref)
    tot = (p_ref[0] + p_ref[1] + g1_ref[...]) * dis
    u = jnp.where(tot >= 0, tot, 0.01 * tot)
    h = lax.dot_general(u, w_ref[...], (((1,), (1,)), ((), ())),
                        preferred_element_type=jnp.float32,
                        precision=lax.Precision.HIGHEST)
    o_ref[...] = h * dis


def _pool_body(p_ref, g2_ref, degp_ref, b_ref, o_ref):
    i = pl.program_id(0)
    dis = _dis_from(degp_ref)
    h2 = (p_ref[0] + p_ref[1] + g2_ref[...]) * dis
    b = b_ref[0, 0, :]
    gids = lax.broadcasted_iota(jnp.int32, (G, BLK), 0)
    sel = (b[None, :] == gids).astype(jnp.float32)
    contrib = lax.dot_general(sel, h2, (((1,), (0,)), ((), ())),
                              preferred_element_type=jnp.float32,
                              precision=lax.Precision.HIGHEST)

    @pl.when(i == 0)
    def _():
        o_ref[...] = contrib

    @pl.when(i > 0)
    def _():
        o_ref[...] += contrib


_g1_call = pl.pallas_call(
    _g1_body,
    grid=(NB,),
    in_specs=[
        pl.BlockSpec((BLK, D), lambda i: (i, 0)),
        pl.BlockSpec((D, D), lambda i: (0, 0)),
        pl.BlockSpec((NC, BLK, 16), lambda i: (0, i, 0)),
    ],
    out_specs=pl.BlockSpec((BLK, D), lambda i: (i, 0)),
    out_shape=jax.ShapeDtypeStruct((N, D), jnp.float32),
)

_g2_call = pl.pallas_call(
    _g2_body,
    grid=(NB,),
    in_specs=[
        pl.BlockSpec((NC, BLK, D), lambda i: (0, i, 0)),
        pl.BlockSpec((BLK, D), lambda i: (i, 0)),
        pl.BlockSpec((NC, BLK, 16), lambda i: (0, i, 0)),
        pl.BlockSpec((D, D), lambda i: (0, 0)),
    ],
    out_specs=pl.BlockSpec((BLK, D), lambda i: (i, 0)),
    out_shape=jax.ShapeDtypeStruct((N, D), jnp.float32),
)

_pool_call = pl.pallas_call(
    _pool_body,
    grid=(NB,),
    in_specs=[
        pl.BlockSpec((NC, BLK, D), lambda i: (0, i, 0)),
        pl.BlockSpec((BLK, D), lambda i: (i, 0)),
        pl.BlockSpec((NC, BLK, 16), lambda i: (0, i, 0)),
        pl.BlockSpec((1, 1, BLK), lambda i: (i, 0, 0)),
    ],
    out_specs=pl.BlockSpec((G, D), lambda i: (0, 0)),
    out_shape=jax.ShapeDtypeStruct((G, D), jnp.float32),
)


def kernel(x, edge_index, batch, W1, W2):
    src = edge_index[0]
    dst = edge_index[1]
    npad = EPAD - E
    src_p = jnp.concatenate([src, jnp.zeros((npad,), jnp.int32)])
    dst_p = jnp.concatenate(
        [dst, N + (jnp.arange(npad, dtype=jnp.int32) % (NP - N))])
    dst2 = dst_p.reshape(NW * NCH, CH)

    onesD = jnp.ones((CH, D), jnp.float32)
    zD = jnp.zeros((N, D), jnp.float32)
    batch3 = batch.reshape(NB, 1, BLK)

    degp_w = _deg_pass(dst2, zD, onesD)
    degp = lax.slice(degp_w, (0, 0, 0), (NC, N, 16))
    g1 = _g1_call(x, W1, degp)
    p1 = _edge_pass(g1, src_p, dst2, zD)
    g2 = _g2_call(p1, g1, degp, W2)
    p2 = _edge_pass(g2, src_p, dst2, zD)
    return _pool_call(p2, g2, degp, batch3)


# speedup vs baseline: 3.0186x; 3.0186x over previous
import functools

import jax
import jax.numpy as jnp
from jax import lax
from jax.experimental import pallas as pl
from jax.experimental.pallas import tpu as pltpu
from jax.experimental.pallas import tpu_sc as plsc

N = 10000
E = 320000
D = 128
G = 64
NC = 2
NS = 16
NW = NC * NS

CH = 128
NCH = 80
NPAIR = NCH // 2
EPW = NCH * CH
EPAD = NW * EPW
NP = 10112

RPT = 624
RTAIL = N - NS * RPT
RTOFF = NS * RPT

BLK = 2000
NB = N // BLK

_mesh = plsc.VectorSubcoreMesh(core_axis_name="c", subcore_axis_name="s")


def _wait_rows(sem, g_hbm, rows):
    pltpu.make_async_copy(g_hbm.at[pl.ds(0, CH)], rows, sem).wait()


@functools.partial(
    pl.kernel,
    out_type=jax.ShapeDtypeStruct((NC, N, D), jnp.float32),
    mesh=_mesh,
    scratch_types=[
        pltpu.VMEM((NCH, CH), jnp.int32),
        pltpu.VMEM((CH, D), jnp.float32),
        pltpu.VMEM_SHARED((NP, D), jnp.float32),
        pltpu.SemaphoreType.DMA,
        pltpu.SemaphoreType.DMA,
    ],
)
def _deg_pass(dst2_hbm, zD_hbm, ones_hbm, out_hbm,
              dstb_v, ones_v, acc_sh, sem_a, sem_b):
    c = lax.axis_index("c")
    s = lax.axis_index("s")
    w = c * NS + s
    rbase = s * RPT
    pltpu.sync_copy(zD_hbm.at[pl.ds(rbase, RPT)], acc_sh.at[pl.ds(rbase, RPT)])

    @pl.when(s == NS - 1)
    def _():
        pltpu.sync_copy(zD_hbm.at[pl.ds(RTOFF, RTAIL)],
                        acc_sh.at[pl.ds(RTOFF, RTAIL)])

    pltpu.sync_copy(ones_hbm, ones_v)
    pltpu.sync_copy(dst2_hbm.at[pl.ds(w * NCH, NCH)], dstb_v)
    plsc.subcore_barrier()

    def body(j, carry):
        @pl.when(j > 0)
        def _():
            _wait_rows(sem_a, zD_hbm, ones_v)
        pltpu.async_copy(ones_v, acc_sh.at[dstb_v.at[2 * j]], sem_a, add=True)

        @pl.when(j > 0)
        def _():
            _wait_rows(sem_b, zD_hbm, ones_v)
        pltpu.async_copy(ones_v, acc_sh.at[dstb_v.at[2 * j + 1]], sem_b,
                         add=True)
        return carry

    lax.fori_loop(0, NPAIR, body, 0)
    _wait_rows(sem_a, zD_hbm, ones_v)
    _wait_rows(sem_b, zD_hbm, ones_v)
    plsc.subcore_barrier()
    pltpu.sync_copy(acc_sh.at[pl.ds(rbase, RPT)], out_hbm.at[c, pl.ds(rbase, RPT)])

    @pl.when(s == NS - 1)
    def _():
        pltpu.sync_copy(acc_sh.at[pl.ds(RTOFF, RTAIL)],
                        out_hbm.at[c, pl.ds(RTOFF, RTAIL)])


@functools.partial(
    pl.kernel,
    out_type=jax.ShapeDtypeStruct((NC, N, D), jnp.float32),
    mesh=_mesh,
    scratch_types=[
        pltpu.VMEM((NCH, CH), jnp.int32),
        pltpu.VMEM((CH,), jnp.int32),
        pltpu.VMEM((CH,), jnp.int32),
        pltpu.VMEM((CH, D), jnp.float32),
        pltpu.VMEM((CH, D), jnp.float32),
        pltpu.VMEM_SHARED((NP, D), jnp.float32),
        pltpu.SemaphoreType.DMA,
        pltpu.SemaphoreType.DMA,
        pltpu.SemaphoreType.DMA,
        pltpu.SemaphoreType.DMA,
        pltpu.SemaphoreType.DMA,
        pltpu.SemaphoreType.DMA,
    ],
)
def _edge_pass(g_hbm, src_hbm, dst2_hbm, zD_hbm, out_hbm,
               dstb_v, srcv_a, srcv_b, rows_a, rows_b, acc_sh,
               isem_a, isem_b, gsem_a, gsem_b, ssem_a, ssem_b):
    c = lax.axis_index("c")
    s = lax.axis_index("s")
    w = c * NS + s
    rbase = s * RPT
    ebase = w * EPW
    pltpu.sync_copy(zD_hbm.at[pl.ds(rbase, RPT)], acc_sh.at[pl.ds(rbase, RPT)])

    @pl.when(s == NS - 1)
    def _():
        pltpu.sync_copy(zD_hbm.at[pl.ds(RTOFF, RTAIL)],
                        acc_sh.at[pl.ds(RTOFF, RTAIL)])

    pltpu.sync_copy(dst2_hbm.at[pl.ds(w * NCH, NCH)], dstb_v)
    plsc.subcore_barrier()

    def idx(chunk, srcv, sem):
        pltpu.async_copy(src_hbm.at[pl.ds(ebase + chunk * CH, CH)], srcv, sem)

    def wait_idx(sem, srcv):
        pltpu.make_async_copy(src_hbm.at[pl.ds(0, CH)], srcv, sem).wait()

    idx(0, srcv_a, isem_a)
    idx(1, srcv_b, isem_b)

    def body(j, carry):
        @pl.when(j > 0)
        def _():
            _wait_rows(ssem_a, g_hbm, rows_a)
        wait_idx(isem_a, srcv_a)
        pltpu.async_copy(g_hbm.at[srcv_a], rows_a, gsem_a)

        @pl.when(j > 0)
        def _():
            _wait_rows(gsem_b, g_hbm, rows_b)
            pltpu.async_copy(rows_b, acc_sh.at[dstb_v.at[2 * j - 1]], ssem_b,
                             add=True)
            idx(2 * j + 1, srcv_b, isem_b)
            _wait_rows(ssem_b, g_hbm, rows_b)

        wait_idx(isem_b, srcv_b)
        pltpu.async_copy(g_hbm.at[srcv_b], rows_b, gsem_b)
        _wait_rows(gsem_a, g_hbm, rows_a)
        pltpu.async_copy(rows_a, acc_sh.at[dstb_v.at[2 * j]], ssem_a, add=True)

        @pl.when(j < NPAIR - 1)
        def _():
            idx(2 * j + 2, srcv_a, isem_a)
        return carry

    lax.fori_loop(0, NPAIR, body, 0)
    _wait_rows(gsem_b, g_hbm, rows_b)
    pltpu.async_copy(rows_b, acc_sh.at[dstb_v.at[NCH - 1]], ssem_b, add=True)
    _wait_rows(ssem_a, g_hbm, rows_a)
    _wait_rows(ssem_b, g_hbm, rows_b)
    plsc.subcore_barrier()
    pltpu.sync_copy(acc_sh.at[pl.ds(rbase, RPT)], out_hbm.at[c, pl.ds(rbase, RPT)])

    @pl.when(s == NS - 1)
    def _():
        pltpu.sync_copy(acc_sh.at[pl.ds(RTOFF, RTAIL)],
                        out_hbm.at[c, pl.ds(RTOFF, RTAIL)])


def _dis_from(degp_ref):
    deg = degp_ref[0, :, 0:1] + degp_ref[1, :, 0:1] + 1.0
    return lax.rsqrt(deg)


def _g1_body(x_ref, w_ref, degp_ref, o_ref):
    dis = _dis_from(degp_ref)
    h = lax.dot_general(x_ref[...], w_ref[...], (((1,), (1,)), ((), ())),
                        preferred_element_type=jnp.float32,
                        precision=lax.Precision.HIGHEST)
    o_ref[...] = h * dis


def _g2_body(p_ref, g1_ref, degp_ref, w_ref, o_ref):
    dis = _dis_from(degp_ref)
    tot = (p_ref[0] + p_ref[1] + g1_ref[...]) * dis
    u = jnp.where(tot >= 0, tot, 0.01 * tot)
    h = lax.dot_general(u, w_ref[...], (((1,), (1,)), ((), ())),
                        preferred_element_type=jnp.float32,
                        precision=lax.Precision.HIGHEST)
    o_ref[...] = h * dis


def _pool_body(p_ref, g2_ref, degp_ref, b_ref, o_ref):
    i = pl.program_id(0)
    dis = _dis_from(degp_ref)
    h2 = (p_ref[0] + p_ref[1] + g2_ref[...]) * dis
    b = b_ref[0, 0, :]
    gids = lax.broadcasted_iota(jnp.int32, (G, BLK), 0)
    sel = (b[None, :] == gids).astype(jnp.float32)
    contrib = lax.dot_general(sel, h2, (((1,), (0,)), ((), ())),
                              preferred_element_type=jnp.float32,
                              precision=lax.Precision.HIGHEST)

    @pl.when(i == 0)
    def _():
        o_ref[...] = contrib

    @pl.when(i > 0)
    def _():
        o_ref[...] += contrib


_g1_call = pl.pallas_call(
    _g1_body,
    grid=(NB,),
    in_specs=[
        pl.BlockSpec((BLK, D), lambda i: (i, 0)),
        pl.BlockSpec((D, D), lambda i: (0, 0)),
        pl.BlockSpec((NC, BLK, 16), lambda i: (0, i, 0)),
    ],
    out_specs=pl.BlockSpec((BLK, D), lambda i: (i, 0)),
    out_shape=jax.ShapeDtypeStruct((N, D), jnp.float32),
)

_g2_call = pl.pallas_call(
    _g2_body,
    grid=(NB,),
    in_specs=[
        pl.BlockSpec((NC, BLK, D), lambda i: (0, i, 0)),
        pl.BlockSpec((BLK, D), lambda i: (i, 0)),
        pl.BlockSpec((NC, BLK, 16), lambda i: (0, i, 0)),
        pl.BlockSpec((D, D), lambda i: (0, 0)),
    ],
    out_specs=pl.BlockSpec((BLK, D), lambda i: (i, 0)),
    out_shape=jax.ShapeDtypeStruct((N, D), jnp.float32),
)

_pool_call = pl.pallas_call(
    _pool_body,
    grid=(NB,),
    in_specs=[
        pl.BlockSpec((NC, BLK, D), lambda i: (0, i, 0)),
        pl.BlockSpec((BLK, D), lambda i: (i, 0)),
        pl.BlockSpec((NC, BLK, 16), lambda i: (0, i, 0)),
        pl.BlockSpec((1, 1, BLK), lambda i: (i, 0, 0)),
    ],
    out_specs=pl.BlockSpec((G, D), lambda i: (0, 0)),
    out_shape=jax.ShapeDtypeStruct((G, D), jnp.float32),
)


def kernel(x, edge_index, batch, W1, W2):
    src = edge_index[0]
    dst = edge_index[1]
    ppw = EPW - E // NW
    psrc = jnp.broadcast_to(jnp.arange(ppw, dtype=jnp.int32)[None], (NW, ppw))
    pdst = jnp.broadcast_to(
        (N + jnp.arange(ppw, dtype=jnp.int32) % (NP - N))[None], (NW, ppw))
    src_p = jnp.concatenate([src.reshape(NW, E // NW), psrc], axis=1).reshape(-1)
    dst_p = jnp.concatenate([dst.reshape(NW, E // NW), pdst], axis=1).reshape(-1)
    dst2 = dst_p.reshape(NW * NCH, CH)

    onesD = jnp.ones((CH, D), jnp.float32)
    zD = jnp.zeros((N, D), jnp.float32)
    batch3 = batch.reshape(NB, 1, BLK)

    degp_w = _deg_pass(dst2, zD, onesD)
    degp = lax.slice(degp_w, (0, 0, 0), (NC, N, 16))
    g1 = _g1_call(x, W1, degp)
    p1 = _edge_pass(g1, src_p, dst2, zD)
    g2 = _g2_call(p1, g1, degp, W2)
    p2 = _edge_pass(g2, src_p, dst2, zD)
    return _pool_call(p2, g2, degp, batch3)


# speedup vs baseline: 3.2122x; 1.0641x over previous
import functools

import jax
import jax.numpy as jnp
from jax import lax
from jax.experimental import pallas as pl
from jax.experimental.pallas import tpu as pltpu
from jax.experimental.pallas import tpu_sc as plsc

N = 10000
E = 320000
D = 128
G = 64
NC = 2
NS = 16
NW = NC * NS

CH = 128
NCH = 80
NPAIR = NCH // 2
EPW = NCH * CH
EPAD = NW * EPW
ECH = 81
EEPW = ECH * CH
NTRI = ECH // 3 - 1
NP = 10112

RPT = 624
RTAIL = N - NS * RPT
RTOFF = NS * RPT

BLK = 2000
NB = N // BLK

_mesh = plsc.VectorSubcoreMesh(core_axis_name="c", subcore_axis_name="s")


def _wait_rows(sem, g_hbm, rows):
    pltpu.make_async_copy(g_hbm.at[pl.ds(0, CH)], rows, sem).wait()


@functools.partial(
    pl.kernel,
    out_type=jax.ShapeDtypeStruct((NC, N, D), jnp.float32),
    mesh=_mesh,
    scratch_types=[
        pltpu.VMEM((NCH, CH), jnp.int32),
        pltpu.VMEM((CH, D), jnp.float32),
        pltpu.VMEM_SHARED((NP, D), jnp.float32),
        pltpu.SemaphoreType.DMA,
        pltpu.SemaphoreType.DMA,
    ],
)
def _deg_pass(dst2_hbm, zD_hbm, ones_hbm, out_hbm,
              dstb_v, ones_v, acc_sh, sem_a, sem_b):
    c = lax.axis_index("c")
    s = lax.axis_index("s")
    w = c * NS + s
    rbase = s * RPT
    pltpu.sync_copy(zD_hbm.at[pl.ds(rbase, RPT)], acc_sh.at[pl.ds(rbase, RPT)])

    @pl.when(s == NS - 1)
    def _():
        pltpu.sync_copy(zD_hbm.at[pl.ds(RTOFF, RTAIL)],
                        acc_sh.at[pl.ds(RTOFF, RTAIL)])

    pltpu.sync_copy(ones_hbm, ones_v)
    pltpu.sync_copy(dst2_hbm.at[pl.ds(w * NCH, NCH)], dstb_v)
    plsc.subcore_barrier()

    def body(j, carry):
        @pl.when(j > 0)
        def _():
            _wait_rows(sem_a, zD_hbm, ones_v)
        pltpu.async_copy(ones_v, acc_sh.at[dstb_v.at[2 * j]], sem_a, add=True)

        @pl.when(j > 0)
        def _():
            _wait_rows(sem_b, zD_hbm, ones_v)
        pltpu.async_copy(ones_v, acc_sh.at[dstb_v.at[2 * j + 1]], sem_b,
                         add=True)
        return carry

    lax.fori_loop(0, NPAIR, body, 0)
    _wait_rows(sem_a, zD_hbm, ones_v)
    _wait_rows(sem_b, zD_hbm, ones_v)
    plsc.subcore_barrier()
    pltpu.sync_copy(acc_sh.at[pl.ds(rbase, RPT)], out_hbm.at[c, pl.ds(rbase, RPT)])

    @pl.when(s == NS - 1)
    def _():
        pltpu.sync_copy(acc_sh.at[pl.ds(RTOFF, RTAIL)],
                        out_hbm.at[c, pl.ds(RTOFF, RTAIL)])


@functools.partial(
    pl.kernel,
    out_type=jax.ShapeDtypeStruct((NC, N, D), jnp.float32),
    mesh=_mesh,
    scratch_types=[
        [pltpu.VMEM((CH,), jnp.int32)] * 3,
        [pltpu.VMEM((CH,), jnp.int32)] * 3,
        [pltpu.VMEM((CH, D), jnp.float32)] * 3,
        pltpu.VMEM_SHARED((NP, D), jnp.float32),
        [pltpu.SemaphoreType.DMA] * 3,
        [pltpu.SemaphoreType.DMA] * 3,
        [pltpu.SemaphoreType.DMA] * 3,
        [pltpu.SemaphoreType.DMA] * 3,
    ],
)
def _edge_pass(g_hbm, src_hbm, dst_hbm, zD_hbm, out_hbm,
               srcv, dstv, rows, acc_sh, isem, dsem, gsem, ssem):
    c = lax.axis_index("c")
    s = lax.axis_index("s")
    w = c * NS + s
    rbase = s * RPT
    ebase = w * EEPW
    pltpu.sync_copy(zD_hbm.at[pl.ds(rbase, RPT)], acc_sh.at[pl.ds(rbase, RPT)])

    @pl.when(s == NS - 1)
    def _():
        pltpu.sync_copy(zD_hbm.at[pl.ds(RTOFF, RTAIL)],
                        acc_sh.at[pl.ds(RTOFF, RTAIL)])

    plsc.subcore_barrier()

    def srcidx(chunk, b):
        pltpu.async_copy(src_hbm.at[pl.ds(ebase + chunk * CH, CH)], srcv[b],
                         isem[b])

    def dstidx(chunk, b):
        pltpu.async_copy(dst_hbm.at[pl.ds(ebase + chunk * CH, CH)], dstv[b],
                         dsem[b])

    def wait_idx(sem, v):
        pltpu.make_async_copy(src_hbm.at[pl.ds(0, CH)], v, sem).wait()

    def gather(chunk_b, b):
        wait_idx(isem[b], srcv[b])
        pltpu.async_copy(g_hbm.at[srcv[b]], rows[b], gsem[b])

    def scatter(b):
        _wait_rows(gsem[b], g_hbm, rows[b])
        wait_idx(dsem[b], dstv[b])
        pltpu.async_copy(rows[b], acc_sh.at[dstv[b]], ssem[b], add=True)

    srcidx(0, 0)
    srcidx(1, 1)
    srcidx(2, 2)
    dstidx(0, 0)
    gather(0, 0)
    dstidx(1, 1)
    gather(1, 1)
    dstidx(2, 2)
    gather(2, 2)
    scatter(0)
    srcidx(3, 0)

    def body(j, carry):
        k = 3 * j + 3

        def step(off, b, b2, last):
            kk = k + off
            _wait_rows(ssem[b], g_hbm, rows[b])
            dstidx(kk, b)
            gather(kk, b)
            scatter(b2)
            if last:
                @pl.when(j < NTRI - 1)
                def _():
                    srcidx(kk + 1, b2)
            else:
                srcidx(kk + 1, b2)

        step(0, 0, 1, False)
        step(1, 1, 2, False)
        step(2, 2, 0, True)
        return carry

    lax.fori_loop(0, NTRI, body, 0)
    scatter(1)
    scatter(2)
    _wait_rows(ssem[0], g_hbm, rows[0])
    _wait_rows(ssem[1], g_hbm, rows[1])
    _wait_rows(ssem[2], g_hbm, rows[2])
    plsc.subcore_barrier()
    pltpu.sync_copy(acc_sh.at[pl.ds(rbase, RPT)], out_hbm.at[c, pl.ds(rbase, RPT)])

    @pl.when(s == NS - 1)
    def _():
        pltpu.sync_copy(acc_sh.at[pl.ds(RTOFF, RTAIL)],
                        out_hbm.at[c, pl.ds(RTOFF, RTAIL)])


def _dis_from(degp_ref):
    deg = degp_ref[0, :, 0:1] + degp_ref[1, :, 0:1] + 1.0
    return lax.rsqrt(deg)


def _g1_body(x_ref, w_ref, degp_ref, o_ref):
    dis = _dis_from(degp_ref)
    h = lax.dot_general(x_ref[...], w_ref[...], (((1,), (1,)), ((), ())),
                        preferred_element_type=jnp.float32,
                        precision=lax.Precision.HIGHEST)
    o_ref[...] = h * dis


def _g2_body(p_ref, g1_ref, degp_ref, w_ref, o_ref):
    dis = _dis_from(degp_ref)
    tot = (p_ref[0] + p_ref[1] + g1_ref[...]) * dis
    u = jnp.where(tot >= 0, tot, 0.01 * tot)
    h = lax.dot_general(u, w_ref[...], (((1,), (1,)), ((), ())),
                        preferred_element_type=jnp.float32,
                        precision=lax.Precision.HIGHEST)
    o_ref[...] = h * dis


def _pool_body(p_ref, g2_ref, degp_ref, b_ref, o_ref):
    i = pl.program_id(0)
    dis = _dis_from(degp_ref)
    h2 = (p_ref[0] + p_ref[1] + g2_ref[...]) * dis
    b = b_ref[0, 0, :]
    gids = lax.broadcasted_iota(jnp.int32, (G, BLK), 0)
    sel = (b[None, :] == gids).astype(jnp.float32)
    contrib = lax.dot_general(sel, h2, (((1,), (0,)), ((), ())),
                              preferred_element_type=jnp.float32,
                              precision=lax.Precision.HIGHEST)

    @pl.when(i == 0)
    def _():
        o_ref[...] = contrib

    @pl.when(i > 0)
    def _():
        o_ref[...] += contrib


_g1_call = pl.pallas_call(
    _g1_body,
    grid=(NB,),
    in_specs=[
        pl.BlockSpec((BLK, D), lambda i: (i, 0)),
        pl.BlockSpec((D, D), lambda i: (0, 0)),
        pl.BlockSpec((NC, BLK, 16), lambda i: (0, i, 0)),
    ],
    out_specs=pl.BlockSpec((BLK, D), lambda i: (i, 0)),
    out_shape=jax.ShapeDtypeStruct((N, D), jnp.float32),
)

_g2_call = pl.pallas_call(
    _g2_body,
    grid=(NB,),
    in_specs=[
        pl.BlockSpec((NC, BLK, D), lambda i: (0, i, 0)),
        pl.BlockSpec((BLK, D), lambda i: (i, 0)),
        pl.BlockSpec((NC, BLK, 16), lambda i: (0, i, 0)),
        pl.BlockSpec((D, D), lambda i: (0, 0)),
    ],
    out_specs=pl.BlockSpec((BLK, D), lambda i: (i, 0)),
    out_shape=jax.ShapeDtypeStruct((N, D), jnp.float32),
)

_pool_call = pl.pallas_call(
    _pool_body,
    grid=(NB,),
    in_specs=[
        pl.BlockSpec((NC, BLK, D), lambda i: (0, i, 0)),
        pl.BlockSpec((BLK, D), lambda i: (i, 0)),
        pl.BlockSpec((NC, BLK, 16), lambda i: (0, i, 0)),
        pl.BlockSpec((1, 1, BLK), lambda i: (i, 0, 0)),
    ],
    out_specs=pl.BlockSpec((G, D), lambda i: (0, 0)),
    out_shape=jax.ShapeDtypeStruct((G, D), jnp.float32),
)


def kernel(x, edge_index, batch, W1, W2):
    src = edge_index[0]
    dst = edge_index[1]
    def pad_edges(arr, per_worker, pad_vals):
        blocks = arr.reshape(NW, E // NW)
        padb = jnp.broadcast_to(pad_vals[None], (NW, per_worker))
        return jnp.concatenate([blocks, padb], axis=1).reshape(-1)

    dpw = EPW - E // NW
    epw = EEPW - E // NW
    spare = lambda n: N + jnp.arange(n, dtype=jnp.int32) % (NP - N)
    dst2 = pad_edges(dst, dpw, spare(dpw)).reshape(NW * NCH, CH)
    src_e = pad_edges(src, epw, jnp.arange(epw, dtype=jnp.int32))
    dst_e = pad_edges(dst, epw, spare(epw))

    onesD = jnp.ones((CH, D), jnp.float32)
    zD = jnp.zeros((N, D), jnp.float32)
    batch3 = batch.reshape(NB, 1, BLK)

    degp_w = _deg_pass(dst2, zD, onesD)
    degp = lax.slice(degp_w, (0, 0, 0), (NC, N, 16))
    g1 = _g1_call(x, W1, degp)
    p1 = _edge_pass(g1, src_e, dst_e, zD)
    g2 = _g2_call(p1, g1, degp, W2)
    p2 = _edge_pass(g2, src_e, dst_e, zD)
    return _pool_call(p2, g2, degp, batch3)


# speedup vs baseline: 3.2163x; 1.0013x over previous
import functools

import jax
import jax.numpy as jnp
from jax import lax
from jax.experimental import pallas as pl
from jax.experimental.pallas import tpu as pltpu
from jax.experimental.pallas import tpu_sc as plsc

N = 10000
E = 320000
D = 128
G = 64
NC = 2
NS = 16
NW = NC * NS

CH = 128
NCH = 80
NPAIR = NCH // 2
EPW = NCH * CH
EPAD = NW * EPW
ECH = 81
EEPW = ECH * CH
NTRI = ECH // 3 - 1
NP = 10112

RPT = 624
RTAIL = N - NS * RPT
RTOFF = NS * RPT

BLK = 2000
NB = N // BLK

_mesh = plsc.VectorSubcoreMesh(core_axis_name="c", subcore_axis_name="s")


def _wait_rows(sem, g_hbm, rows):
    pltpu.make_async_copy(g_hbm.at[pl.ds(0, CH)], rows, sem).wait()


@functools.partial(
    pl.kernel,
    out_type=jax.ShapeDtypeStruct((NC, N, D), jnp.float32),
    mesh=_mesh,
    scratch_types=[
        pltpu.VMEM((NCH, CH), jnp.int32),
        pltpu.VMEM((CH, D), jnp.float32),
        pltpu.VMEM_SHARED((NP, D), jnp.float32),
        pltpu.SemaphoreType.DMA,
        pltpu.SemaphoreType.DMA,
    ],
)
def _deg_pass(dst2_hbm, zD_hbm, ones_hbm, out_hbm,
              dstb_v, ones_v, acc_sh, sem_a, sem_b):
    c = lax.axis_index("c")
    s = lax.axis_index("s")
    w = c * NS + s
    rbase = s * RPT
    pltpu.sync_copy(zD_hbm.at[pl.ds(rbase, RPT)], acc_sh.at[pl.ds(rbase, RPT)])

    @pl.when(s == NS - 1)
    def _():
        pltpu.sync_copy(zD_hbm.at[pl.ds(RTOFF, RTAIL)],
                        acc_sh.at[pl.ds(RTOFF, RTAIL)])

    pltpu.sync_copy(ones_hbm, ones_v)
    pltpu.sync_copy(dst2_hbm.at[pl.ds(w * NCH, NCH)], dstb_v)
    plsc.subcore_barrier()

    def body(j, carry):
        @pl.when(j > 0)
        def _():
            _wait_rows(sem_a, zD_hbm, ones_v)
        pltpu.async_copy(ones_v, acc_sh.at[dstb_v.at[2 * j]], sem_a, add=True)

        @pl.when(j > 0)
        def _():
            _wait_rows(sem_b, zD_hbm, ones_v)
        pltpu.async_copy(ones_v, acc_sh.at[dstb_v.at[2 * j + 1]], sem_b,
                         add=True)
        return carry

    lax.fori_loop(0, NPAIR, body, 0)
    _wait_rows(sem_a, zD_hbm, ones_v)
    _wait_rows(sem_b, zD_hbm, ones_v)
    plsc.subcore_barrier()
    pltpu.sync_copy(acc_sh.at[pl.ds(rbase, RPT)], out_hbm.at[c, pl.ds(rbase, RPT)])

    @pl.when(s == NS - 1)
    def _():
        pltpu.sync_copy(acc_sh.at[pl.ds(RTOFF, RTAIL)],
                        out_hbm.at[c, pl.ds(RTOFF, RTAIL)])


@functools.partial(
    pl.kernel,
    out_type=jax.ShapeDtypeStruct((NC, N, D), jnp.float32),
    mesh=_mesh,
    scratch_types=[
        [pltpu.VMEM((CH,), jnp.int32)] * 3,
        [pltpu.VMEM((CH,), jnp.int32)] * 3,
        [pltpu.VMEM((CH, D), jnp.float32)] * 3,
        pltpu.VMEM_SHARED((NP, D), jnp.float32),
        [pltpu.SemaphoreType.DMA] * 3,
        [pltpu.SemaphoreType.DMA] * 3,
        [pltpu.SemaphoreType.DMA] * 3,
        [pltpu.SemaphoreType.DMA] * 3,
    ],
)
def _edge_pass(g_hbm, src_hbm, dst_hbm, zD_hbm, out_hbm,
               srcv, dstv, rows, acc_sh, isem, dsem, gsem, ssem):
    c = lax.axis_index("c")
    s = lax.axis_index("s")
    w = c * NS + s
    rbase = s * RPT
    ebase = w * EEPW
    pltpu.sync_copy(zD_hbm.at[pl.ds(rbase, RPT)], acc_sh.at[pl.ds(rbase, RPT)])

    @pl.when(s == NS - 1)
    def _():
        pltpu.sync_copy(zD_hbm.at[pl.ds(RTOFF, RTAIL)],
                        acc_sh.at[pl.ds(RTOFF, RTAIL)])

    plsc.subcore_barrier()

    def srcidx(chunk, b):
        pltpu.async_copy(src_hbm.at[pl.ds(ebase + chunk * CH, CH)], srcv[b],
                         isem[b])

    def dstidx(chunk, b):
        pltpu.async_copy(dst_hbm.at[pl.ds(ebase + chunk * CH, CH)], dstv[b],
                         dsem[b])

    def wait_idx(sem, v):
        pltpu.make_async_copy(src_hbm.at[pl.ds(0, CH)], v, sem).wait()

    def gather(chunk_b, b):
        wait_idx(isem[b], srcv[b])
        pltpu.async_copy(g_hbm.at[srcv[b]], rows[b], gsem[b])

    def scatter(b):
        _wait_rows(gsem[b], g_hbm, rows[b])
        wait_idx(dsem[b], dstv[b])
        pltpu.async_copy(rows[b], acc_sh.at[dstv[b]], ssem[b], add=True)

    srcidx(0, 0)
    srcidx(1, 1)
    srcidx(2, 2)
    dstidx(0, 0)
    gather(0, 0)
    dstidx(1, 1)
    gather(1, 1)
    dstidx(2, 2)
    gather(2, 2)
    scatter(0)
    srcidx(3, 0)

    def body(j, carry):
        k = 3 * j + 3

        def step(off, b, b2, last):
            kk = k + off
            _wait_rows(ssem[b], g_hbm, rows[b])
            dstidx(kk, b)
            gather(kk, b)
            scatter(b2)
            if last:
                @pl.when(j < NTRI - 1)
                def _():
                    srcidx(kk + 1, b2)
            else:
                srcidx(kk + 1, b2)

        step(0, 0, 1, False)
        step(1, 1, 2, False)
        step(2, 2, 0, True)
        return carry

    lax.fori_loop(0, NTRI, body, 0)
    scatter(1)
    scatter(2)
    _wait_rows(ssem[0], g_hbm, rows[0])
    _wait_rows(ssem[1], g_hbm, rows[1])
    _wait_rows(ssem[2], g_hbm, rows[2])
    plsc.subcore_barrier()
    pltpu.sync_copy(acc_sh.at[pl.ds(rbase, RPT)], out_hbm.at[c, pl.ds(rbase, RPT)])

    @pl.when(s == NS - 1)
    def _():
        pltpu.sync_copy(acc_sh.at[pl.ds(RTOFF, RTAIL)],
                        out_hbm.at[c, pl.ds(RTOFF, RTAIL)])


def _dis_from(degp_ref):
    deg = degp_ref[0, :, 0:1] + degp_ref[1, :, 0:1] + 1.0
    return lax.rsqrt(deg)


def _h1_body(x_ref, w_ref, o_ref):
    o_ref[...] = lax.dot_general(x_ref[...], w_ref[...],
                                 (((1,), (1,)), ((), ())),
                                 preferred_element_type=jnp.float32,
                                 precision=lax.Precision.HIGHEST)


def _scale_body(h_ref, degp_ref, o_ref):
    o_ref[...] = h_ref[...] * _dis_from(degp_ref)


def _g2_body(p_ref, g1_ref, degp_ref, w_ref, o_ref):
    dis = _dis_from(degp_ref)
    tot = (p_ref[0] + p_ref[1] + g1_ref[...]) * dis
    u = jnp.where(tot >= 0, tot, 0.01 * tot)
    h = lax.dot_general(u, w_ref[...], (((1,), (1,)), ((), ())),
                        preferred_element_type=jnp.float32,
                        precision=lax.Precision.HIGHEST)
    o_ref[...] = h * dis


def _pool_body(p_ref, g2_ref, degp_ref, b_ref, o_ref):
    i = pl.program_id(0)
    dis = _dis_from(degp_ref)
    h2 = (p_ref[0] + p_ref[1] + g2_ref[...]) * dis
    b = b_ref[0, 0, :]
    gids = lax.broadcasted_iota(jnp.int32, (G, BLK), 0)
    sel = (b[None, :] == gids).astype(jnp.float32)
    contrib = lax.dot_general(sel, h2, (((1,), (0,)), ((), ())),
                              preferred_element_type=jnp.float32,
                              precision=lax.Precision.HIGHEST)

    @pl.when(i == 0)
    def _():
        o_ref[...] = contrib

    @pl.when(i > 0)
    def _():
        o_ref[...] += contrib


_h1_call = pl.pallas_call(
    _h1_body,
    grid=(NB,),
    in_specs=[
        pl.BlockSpec((BLK, D), lambda i: (i, 0)),
        pl.BlockSpec((D, D), lambda i: (0, 0)),
    ],
    out_specs=pl.BlockSpec((BLK, D), lambda i: (i, 0)),
    out_shape=jax.ShapeDtypeStruct((N, D), jnp.float32),
)

_scale_call = pl.pallas_call(
    _scale_body,
    grid=(NB,),
    in_specs=[
        pl.BlockSpec((BLK, D), lambda i: (i, 0)),
        pl.BlockSpec((NC, BLK, 16), lambda i: (0, i, 0)),
    ],
    out_specs=pl.BlockSpec((BLK, D), lambda i: (i, 0)),
    out_shape=jax.ShapeDtypeStruct((N, D), jnp.float32),
)

_g2_call = pl.pallas_call(
    _g2_body,
    grid=(NB,),
    in_specs=[
        pl.BlockSpec((NC, BLK, D), lambda i: (0, i, 0)),
        pl.BlockSpec((BLK, D), lambda i: (i, 0)),
        pl.BlockSpec((NC, BLK, 16), lambda i: (0, i, 0)),
        pl.BlockSpec((D, D), lambda i: (0, 0)),
    ],
    out_specs=pl.BlockSpec((BLK, D), lambda i: (i, 0)),
    out_shape=jax.ShapeDtypeStruct((N, D), jnp.float32),
)

_pool_call = pl.pallas_call(
    _pool_body,
    grid=(NB,),
    in_specs=[
        pl.BlockSpec((NC, BLK, D), lambda i: (0, i, 0)),
        pl.BlockSpec((BLK, D), lambda i: (i, 0)),
        pl.BlockSpec((NC, BLK, 16), lambda i: (0, i, 0)),
        pl.BlockSpec((1, 1, BLK), lambda i: (i, 0, 0)),
    ],
    out_specs=pl.BlockSpec((G, D), lambda i: (0, 0)),
    out_shape=jax.ShapeDtypeStruct((G, D), jnp.float32),
)


def kernel(x, edge_index, batch, W1, W2):
    src = edge_index[0]
    dst = edge_index[1]
    def pad_edges(arr, per_worker, pad_vals):
        blocks = arr.reshape(NW, E // NW)
        padb = jnp.broadcast_to(pad_vals[None], (NW, per_worker))
        return jnp.concatenate([blocks, padb], axis=1).reshape(-1)

    dpw = EPW - E // NW
    epw = EEPW - E // NW
    spare = lambda n: N + jnp.arange(n, dtype=jnp.int32) % (NP - N)
    dst2 = pad_edges(dst, dpw, spare(dpw)).reshape(NW * NCH, CH)
    src_e = pad_edges(src, epw, jnp.arange(epw, dtype=jnp.int32))
    dst_e = pad_edges(dst, epw, spare(epw))

    onesD = jnp.ones((CH, D), jnp.float32)
    zD = jnp.zeros((N, D), jnp.float32)
    batch3 = batch.reshape(NB, 1, BLK)

    h1 = _h1_call(x, W1)
    degp_w = _deg_pass(dst2, zD, onesD)
    degp = lax.slice(degp_w, (0, 0, 0), (NC, N, 16))
    g1 = _scale_call(h1, degp)
    p1 = _edge_pass(g1, src_e, dst_e, zD)
    g2 = _g2_call(p1, g1, degp, W2)
    p2 = _edge_pass(g2, src_e, dst_e, zD)
    return _pool_call(p2, g2, degp, batch3)
